# baseline (speedup 1.0000x reference)
import functools

import jax
import jax.numpy as jnp
from jax import lax
from jax.experimental import pallas as pl
from jax.experimental.pallas import tpu as pltpu
from jax.experimental.pallas import tpu_sc as plsc

_RATE = 0.8
_ALPHA = 0.8
_BETA = 1.2

_C_SC = 32


def _tc_sum_body(x_ref, o_ref, acc_ref):
    j = pl.program_id(1)
    nj = pl.num_programs(1)

    @pl.when(j == 0)
    def _():
        acc_ref[...] = jnp.zeros_like(acc_ref)

    acc_ref[...] += jnp.sum(x_ref[...], axis=(2, 3))

    @pl.when(j == nj - 1)
    def _():
        o_ref[...] = acc_ref[...][None]


def _mask_body(fv_ref, o_ref, *, k):
    fv = fv_ref[...]
    gt = (fv[:, None, :] > fv[:, :, None]).astype(jnp.float32)
    cnt = jnp.sum(gt, axis=2)
    o_ref[...] = jnp.where(cnt < k, _ALPHA, _BETA).astype(jnp.float32)


def _sc_sums(inputs, c0, c_sc):
    b, c, h, w = inputs.shape
    planes = b * c_sc
    mesh = plsc.VectorSubcoreMesh(core_axis_name="c", subcore_axis_name="s")
    info = plsc.get_sparse_core_info()
    nw = info.num_cores * info.num_subcores
    ppw = planes // nw
    groups = (h * w) // 16

    @functools.partial(
        pl.kernel,
        mesh=mesh,
        out_type=jax.ShapeDtypeStruct((planes, 16), jnp.float32),
        scratch_types=[
            pltpu.VMEM((h, w), jnp.float32),
            pltpu.VMEM((16,), jnp.float32),
            pltpu.VMEM((ppw, 16), jnp.float32),
        ],
    )
    def sc_kernel(x_hbm, o_hbm, buf, acc, res):
        wid = lax.axis_index("s") * info.num_cores + lax.axis_index("c")
        base = wid * ppw

        @pl.loop(0, ppw)
        def _(i):
            plane = base + i
            bi = plane // c_sc
            ch = c0 + lax.rem(plane, c_sc)
            pltpu.sync_copy(x_hbm.at[bi, ch], buf)
            acc[...] = jnp.zeros((16,), jnp.float32)

            @pl.loop(0, h)
            def _(r):
                for g in range(w // 16):
                    acc[...] += buf[r, pl.ds(g * 16, 16)]

            res[i, :] = acc[...]

        pltpu.sync_copy(res, o_hbm.at[pl.ds(base, ppw)])

    return sc_kernel(inputs)


def kernel(inputs):
    b, c, h, w = inputs.shape
    n = h * w
    k = int(c * _RATE)
    c_sc = _C_SC
    c_tc = c - c_sc

    hb = 56
    steps = h // hb

    sums_tc = pl.pallas_call(
        _tc_sum_body,
        grid=(b, steps),
        in_specs=[pl.BlockSpec((1, c_tc, hb, w), lambda i, j: (i, 0, j, 0))],
        out_specs=pl.BlockSpec((1, 1, c_tc), lambda i, j: (i, 0, 0)),
        out_shape=jax.ShapeDtypeStruct((b, 1, c_tc), jnp.float32),
        scratch_shapes=[pltpu.VMEM((1, c_tc), jnp.float32)],
        compiler_params=pltpu.CompilerParams(
            dimension_semantics=("parallel", "arbitrary"),
        ),
    )(inputs)

    sums_sc = _sc_sums(inputs, c_tc, c_sc).sum(axis=1).reshape(b, c_sc)

    fv = jnp.concatenate([sums_tc.reshape(b, c_tc), sums_sc], axis=1) * (
        1.0 / n
    )
    return pl.pallas_call(
        functools.partial(_mask_body, k=k),
        out_shape=jax.ShapeDtypeStruct((b, c), jnp.float32),
    )(fv)


# speedup vs baseline: 1.3661x; 1.3661x over previous
import functools

import jax
import jax.numpy as jnp
from jax import lax
from jax.experimental import pallas as pl
from jax.experimental.pallas import tpu as pltpu
from jax.experimental.pallas import tpu_sc as plsc

_RATE = 0.8
_ALPHA = 0.8
_BETA = 1.2

_C_SC = 32


def _tc_sum_body(x_ref, o_ref, acc_ref):
    j = pl.program_id(1)
    nj = pl.num_programs(1)

    @pl.when(j == 0)
    def _():
        acc_ref[...] = jnp.zeros_like(acc_ref)

    acc_ref[...] += jnp.sum(x_ref[...], axis=(2, 3))

    @pl.when(j == nj - 1)
    def _():
        o_ref[...] = acc_ref[...][None]


def _mask_body(fv_ref, o_ref, *, k):
    fv = fv_ref[...]
    gt = (fv[:, None, :] > fv[:, :, None]).astype(jnp.float32)
    cnt = jnp.sum(gt, axis=2)
    o_ref[...] = jnp.where(cnt < k, _ALPHA, _BETA).astype(jnp.float32)


def _sc_sums(inputs, c0, c_sc):
    b, c, h, w = inputs.shape
    planes = b * c_sc
    mesh = plsc.VectorSubcoreMesh(core_axis_name="c", subcore_axis_name="s")
    info = plsc.get_sparse_core_info()
    nw = info.num_cores * info.num_subcores
    ppw = planes // nw
    groups = (h * w) // 16

    ng = w // 16

    @functools.partial(
        pl.kernel,
        mesh=mesh,
        out_type=jax.ShapeDtypeStruct((planes, 16), jnp.float32),
        scratch_types=[
            pltpu.VMEM((h, w), jnp.float32),
            pltpu.VMEM((h, w), jnp.float32),
            pltpu.VMEM((ppw, 16), jnp.float32),
            pltpu.SemaphoreType.DMA,
            pltpu.SemaphoreType.DMA,
        ],
    )
    def sc_kernel(x_hbm, o_hbm, buf0, buf1, res, sem0, sem1):
        wid = lax.axis_index("s") * info.num_cores + lax.axis_index("c")
        base = wid * ppw
        bufs = (buf0, buf1)
        sems = (sem0, sem1)

        def plane_src(i):
            plane = base + i
            return x_hbm.at[plane // c_sc, c0 + lax.rem(plane, c_sc)]

        copies = [None] * ppw
        copies[0] = pltpu.make_async_copy(plane_src(0), bufs[0], sems[0])
        copies[0].start()
        for i in range(ppw):
            if i + 1 < ppw:
                copies[i + 1] = pltpu.make_async_copy(
                    plane_src(i + 1), bufs[(i + 1) % 2], sems[(i + 1) % 2]
                )
                copies[i + 1].start()
            copies[i].wait()
            buf = bufs[i % 2]

            def row_body(r, accs):
                return tuple(
                    accs[g] + buf[r, pl.ds(g * 16, 16)] for g in range(ng)
                )

            accs = lax.fori_loop(
                0,
                h,
                row_body,
                tuple(jnp.zeros((16,), jnp.float32) for _ in range(ng)),
            )
            while len(accs) > 1:
                accs = tuple(
                    accs[2 * t] + accs[2 * t + 1]
                    if 2 * t + 1 < len(accs)
                    else accs[2 * t]
                    for t in range((len(accs) + 1) // 2)
                )
            res[i, :] = accs[0]

        pltpu.sync_copy(res, o_hbm.at[pl.ds(base, ppw)])

    return sc_kernel(inputs)


def kernel(inputs):
    b, c, h, w = inputs.shape
    n = h * w
    k = int(c * _RATE)
    c_sc = _C_SC
    c_tc = c - c_sc

    hb = 56
    steps = h // hb

    sums_tc = pl.pallas_call(
        _tc_sum_body,
        grid=(b, steps),
        in_specs=[pl.BlockSpec((1, c_tc, hb, w), lambda i, j: (i, 0, j, 0))],
        out_specs=pl.BlockSpec((1, 1, c_tc), lambda i, j: (i, 0, 0)),
        out_shape=jax.ShapeDtypeStruct((b, 1, c_tc), jnp.float32),
        scratch_shapes=[pltpu.VMEM((1, c_tc), jnp.float32)],
        compiler_params=pltpu.CompilerParams(
            dimension_semantics=("parallel", "arbitrary"),
        ),
    )(inputs)

    sums_sc = _sc_sums(inputs, c_tc, c_sc).sum(axis=1).reshape(b, c_sc)

    fv = jnp.concatenate([sums_tc.reshape(b, c_tc), sums_sc], axis=1) * (
        1.0 / n
    )
    return pl.pallas_call(
        functools.partial(_mask_body, k=k),
        out_shape=jax.ShapeDtypeStruct((b, c), jnp.float32),
    )(fv)


# speedup vs baseline: 1.6461x; 1.2050x over previous
import functools

import jax
import jax.numpy as jnp
from jax.experimental import pallas as pl
from jax.experimental.pallas import tpu as pltpu

_RATE = 0.8
_ALPHA = 0.8
_BETA = 1.2


def _body(x0_ref, x1_ref, o_ref, acc0_ref, acc1_ref, *, k, inv_n):
    j = pl.program_id(1)
    nj = pl.num_programs(1)

    @pl.when(j == 0)
    def _():
        acc0_ref[...] = jnp.zeros_like(acc0_ref)
        acc1_ref[...] = jnp.zeros_like(acc1_ref)

    acc0_ref[...] += jnp.sum(x0_ref[...], axis=(2, 3))
    acc1_ref[...] += jnp.sum(x1_ref[...], axis=(2, 3))

    @pl.when(j == nj - 1)
    def _():
        fv = jnp.concatenate([acc0_ref[...], acc1_ref[...]], axis=1) * inv_n
        gt = (fv[:, None, :] > fv[:, :, None]).astype(jnp.float32)
        cnt = jnp.sum(gt, axis=2)
        o_ref[...] = jnp.where(cnt < k, _ALPHA, _BETA)[None].astype(
            jnp.float32
        )


def kernel(inputs):
    b, c, h, w = inputs.shape
    n = h * w
    k = int(c * _RATE)
    ch = c // 2

    hb = 56
    steps = h // hb

    body = functools.partial(_body, k=k, inv_n=1.0 / n)
    out = pl.pallas_call(
        body,
        grid=(b, steps),
        in_specs=[
            pl.BlockSpec((1, ch, hb, w), lambda i, j: (i, 0, j, 0)),
            pl.BlockSpec((1, ch, hb, w), lambda i, j: (i, 1, j, 0)),
        ],
        out_specs=pl.BlockSpec((1, 1, c), lambda i, j: (i, 0, 0)),
        out_shape=jax.ShapeDtypeStruct((b, 1, c), jnp.float32),
        scratch_shapes=[
            pltpu.VMEM((1, ch), jnp.float32),
            pltpu.VMEM((1, ch), jnp.float32),
        ],
        compiler_params=pltpu.CompilerParams(
            dimension_semantics=("parallel", "arbitrary"),
        ),
    )(inputs, inputs)
    return out.reshape(b, c)


# speedup vs baseline: 1.6479x; 1.0011x over previous
import functools

import jax
import jax.numpy as jnp
from jax.experimental import pallas as pl
from jax.experimental.pallas import tpu as pltpu

_RATE = 0.8
_ALPHA = 0.8
_BETA = 1.2


def _body(x_ref, o_ref, acc_ref, *, k, inv_n):
    j = pl.program_id(1)
    nj = pl.num_programs(1)

    @pl.when(j == 0)
    def _():
        acc_ref[...] = jnp.zeros_like(acc_ref)

    acc_ref[...] += jnp.sum(x_ref[...], axis=(2, 3))

    @pl.when(j == nj - 1)
    def _():
        fv = acc_ref[...] * inv_n
        gt = (fv[:, None, :] > fv[:, :, None]).astype(jnp.float32)
        cnt = jnp.sum(gt, axis=2)
        o_ref[...] = jnp.where(cnt < k, _ALPHA, _BETA)[None].astype(
            jnp.float32
        )


def kernel(inputs):
    b, c, h, w = inputs.shape
    n = h * w
    k = int(c * _RATE)

    hb = 56
    steps = h // hb

    body = functools.partial(_body, k=k, inv_n=1.0 / n)
    out = pl.pallas_call(
        body,
        grid=(b, steps),
        in_specs=[pl.BlockSpec((1, c, hb, w), lambda i, j: (i, 0, j, 0))],
        out_specs=pl.BlockSpec((1, 1, c), lambda i, j: (i, 0, 0)),
        out_shape=jax.ShapeDtypeStruct((b, 1, c), jnp.float32),
        scratch_shapes=[pltpu.VMEM((1, c), jnp.float32)],
        compiler_params=pltpu.CompilerParams(
            dimension_semantics=("parallel", "arbitrary"),
        ),
    )(inputs)
    return out.reshape(b, c)
